# baseline (speedup 1.0000x reference)
import functools
import jax
import jax.numpy as jnp
from jax import lax
from jax.experimental import pallas as pl
from jax.experimental.pallas import tpu as pltpu
from jax.experimental.pallas import tpu_sc as plsc

VOCAB = 50257
N_POS = 1024
D = 768
B = 64
S = 1024

NC = 2
NS = 16
NW = NC * NS
LANES = 16
D_SLICES = D // LANES

SC_W = S // NW
RS = 16
SUB = SC_W // RS
NB = 8
DEPTH = 4
STEPS = B * SUB
UNROLL = NB


def _body(ids_hbm, wte_hbm, wpe_hbm, out_hbm, idx_v, wpe_v, *scratch):
    rows = scratch[:NB]
    gsem = scratch[NB:2 * NB]
    ssem = scratch[2 * NB:3 * NB]
    isem = scratch[3 * NB]

    cid = lax.axis_index("c")
    sid = lax.axis_index("s")
    wid = sid * NC + cid
    s0 = wid * SC_W

    for b in range(B):
        pltpu.async_copy(ids_hbm.at[pl.ds(b * S + s0, SC_W)], idx_v.at[b],
                         isem)
    for b in range(B):
        pltpu.make_async_copy(ids_hbm.at[pl.ds(b * S + s0, SC_W)],
                              idx_v.at[b], isem).wait()
    pltpu.sync_copy(wpe_hbm.at[pl.ds(s0, SC_W), :], wpe_v)

    def idx_slice(i):
        b = lax.shift_right_logical(i, 1)
        h = lax.bitwise_and(i, 1)
        return idx_v.at[b, pl.ds(h * RS, RS)]

    def out_slice(i):
        b = lax.shift_right_logical(i, 1)
        h = lax.bitwise_and(i, 1)
        return out_hbm.at[b, pl.ds(s0 + h * RS, RS), :]

    def issue_gather(i, slot):
        pltpu.async_copy(wte_hbm.at[idx_slice(i)], rows[slot], gsem[slot])

    def wait_gather(i, slot):
        pltpu.make_async_copy(
            wte_hbm.at[idx_slice(i)], rows[slot], gsem[slot]).wait()

    def issue_store(i, slot):
        pltpu.async_copy(rows[slot], out_slice(i), ssem[slot])

    def wait_store(i, slot):
        pltpu.make_async_copy(rows[slot], out_slice(i), ssem[slot]).wait()

    def add_wpe(i, slot):
        h = lax.bitwise_and(i, 1)

        @plsc.parallel_loop(0, RS, 1, unroll=1)
        def _(r):
            for c in range(D_SLICES):
                sl = pl.ds(c * LANES, LANES)
                plsc.addupdate(rows[slot].at[r, sl], wpe_v[h * RS + r, sl])

    def step(i, slot):
        far = (slot + DEPTH) % NB

        @pl.when(i >= DEPTH)
        def _():
            wait_store(i - DEPTH, far)

        @pl.when(i + DEPTH < STEPS)
        def _():
            issue_gather(i + DEPTH, far)
        wait_gather(i, slot)
        add_wpe(i, slot)
        issue_store(i, slot)

    for p in range(DEPTH):
        issue_gather(jnp.int32(p), p)

    def loop_body(j, _):
        i = UNROLL * j
        for r in range(UNROLL):
            step(i + r, r)
        return _

    lax.fori_loop(0, STEPS // UNROLL, loop_body, None)
    for d in range(DEPTH):
        i = STEPS - DEPTH + d
        wait_store(i, i % NB)


@jax.jit
def _embed(input_ids, wte, wpe):
    mesh = plsc.VectorSubcoreMesh(core_axis_name="c", subcore_axis_name="s")
    return pl.kernel(
        _body,
        out_type=jax.ShapeDtypeStruct((B, S, D), jnp.float32),
        mesh=mesh,
        scratch_types=(
            [pltpu.VMEM((B, SC_W), jnp.int32),
             pltpu.VMEM((SC_W, D), jnp.float32)]
            + [pltpu.VMEM((RS, D), jnp.float32)] * NB
            + [pltpu.SemaphoreType.DMA] * (2 * NB + 1)
        ),
    )(input_ids.reshape(-1), wte, wpe)


def kernel(input_ids, attention_mask, wte, wpe):
    del attention_mask
    return _embed(input_ids, wte, wpe)


# speedup vs baseline: 1.0264x; 1.0264x over previous
import functools
import jax
import jax.numpy as jnp
from jax import lax
from jax.experimental import pallas as pl
from jax.experimental.pallas import tpu as pltpu
from jax.experimental.pallas import tpu_sc as plsc

VOCAB = 50257
N_POS = 1024
D = 768
B = 64
S = 1024

NC = 2
NS = 16
NW = NC * NS
LANES = 16
D_SLICES = D // LANES

SC_W = S // NW
NB = 4
DEPTH = 2
STEPS = B
UNROLL = NB


def _body(ids_hbm, wte_hbm, wpe_hbm, out_hbm, idx_v, wpe_v, *scratch):
    rows = scratch[:NB]
    gsem = scratch[NB:2 * NB]
    ssem = scratch[2 * NB:3 * NB]
    isem = scratch[3 * NB]
    wsem = scratch[3 * NB + 1]
    esem = scratch[3 * NB + 2:3 * NB + 2 + DEPTH]

    cid = lax.axis_index("c")
    sid = lax.axis_index("s")
    wid = sid * NC + cid
    s0 = wid * SC_W

    def id_copy(b, sem):
        return pltpu.make_async_copy(ids_hbm.at[pl.ds(b * S + s0, SC_W)],
                                     idx_v.at[b], sem)

    def idx_slice(i):
        return idx_v.at[i]

    def out_slice(i):
        return out_hbm.at[i, pl.ds(s0, SC_W), :]

    def issue_gather(i, slot):
        pltpu.async_copy(wte_hbm.at[idx_slice(i)], rows[slot], gsem[slot])

    def wait_gather(i, slot):
        pltpu.make_async_copy(
            wte_hbm.at[idx_slice(i)], rows[slot], gsem[slot]).wait()

    def issue_store(i, slot):
        pltpu.async_copy(rows[slot], out_slice(i), ssem[slot])

    def wait_store(i, slot):
        pltpu.make_async_copy(rows[slot], out_slice(i), ssem[slot]).wait()

    def add_wpe(i, slot):
        @plsc.parallel_loop(0, SC_W, 1, unroll=1)
        def _(r):
            for c in range(D_SLICES):
                sl = pl.ds(c * LANES, LANES)
                plsc.addupdate(rows[slot].at[r, sl], wpe_v[r, sl])

    def step(i, slot):
        far = (slot + DEPTH) % NB

        @pl.when(i >= DEPTH)
        def _():
            wait_store(i - DEPTH, far)

        @pl.when(i + DEPTH < STEPS)
        def _():
            issue_gather(i + DEPTH, far)
        wait_gather(i, slot)
        add_wpe(i, slot)
        issue_store(i, slot)

    wpe_cp = pltpu.make_async_copy(wpe_hbm.at[pl.ds(s0, SC_W), :], wpe_v,
                                   wsem)
    for p in range(DEPTH):
        id_copy(p, esem[p]).start()
    wpe_cp.start()
    for p in range(DEPTH):
        id_copy(p, esem[p]).wait()
        issue_gather(jnp.int32(p), p)
    for b in range(DEPTH, B):
        id_copy(b, isem).start()
    for b in range(DEPTH, B):
        id_copy(b, isem).wait()
    wpe_cp.wait()

    def loop_body(j, _):
        i = UNROLL * j
        for r in range(UNROLL):
            step(i + r, r)
        return _

    lax.fori_loop(0, STEPS // UNROLL, loop_body, None)
    for d in range(DEPTH):
        i = STEPS - DEPTH + d
        wait_store(i, i % NB)


@jax.jit
def _embed(input_ids, wte, wpe):
    mesh = plsc.VectorSubcoreMesh(core_axis_name="c", subcore_axis_name="s")
    return pl.kernel(
        _body,
        out_type=jax.ShapeDtypeStruct((B, S, D), jnp.float32),
        mesh=mesh,
        scratch_types=(
            [pltpu.VMEM((B, SC_W), jnp.int32),
             pltpu.VMEM((SC_W, D), jnp.float32)]
            + [pltpu.VMEM((SC_W, D), jnp.float32)] * NB
            + [pltpu.SemaphoreType.DMA] * (2 * NB + 2 + DEPTH)
        ),
    )(input_ids.reshape(-1), wte, wpe)


def kernel(input_ids, attention_mask, wte, wpe):
    del attention_mask
    return _embed(input_ids, wte, wpe)
